# baseline (speedup 1.0000x reference)
import functools

import jax
import jax.numpy as jnp
from jax import lax
from jax.experimental import pallas as pl
from jax.experimental.pallas import tpu as pltpu
from jax.experimental.pallas import tpu_sc as plsc

T = 8192
D = 1024
GATE_D = 512
ROLE_D = 192
OCC_D = 64
PARAM_D = 256
EMB_D = GATE_D + ROLE_D + OCC_D
N_GATE = 20
N_ROLE = 4
N_OCC = 2

NC = 2
NS = 16
NW = NC * NS
L = 16

ROWS_PER_W = T // NW
C = 32
N_CHUNKS = ROWS_PER_W // C


def _bcast_i32(val):
    return jnp.full((L,), val, jnp.int32)


def _sc_body(x_hbm, comb_hbm, pos_hbm, w_hbm, b_hbm,
             out_hbm, out_v, g_v, x_v, idx_v, w_v, b_v, psem, gsem):
    cid = lax.axis_index("c")
    sid = lax.axis_index("s")
    wid = sid * NC + cid
    base = wid * ROWS_PER_W

    pltpu.sync_copy(w_hbm, w_v)
    pltpu.sync_copy(b_hbm, b_v)

    iota = lax.iota(jnp.int32, L)

    for k in range(N_CHUNKS):
        rb = base + k * C
        pos_cp = pltpu.async_copy(pos_hbm.at[pl.ds(rb, C)], out_v, psem)
        pltpu.sync_copy(x_hbm.at[pl.ds(rb, C)], x_v)

        for j in range(C // L):
            rows = iota + (j * L)
            g = plsc.load_gather(x_v, [rows, _bcast_i32(0)]).astype(jnp.int32)
            r = plsc.load_gather(x_v, [rows, _bcast_i32(1)]).astype(jnp.int32)
            o = plsc.load_gather(x_v, [rows, _bcast_i32(2)]).astype(jnp.int32)
            idx_v[pl.ds(j * L, L)] = g * (N_ROLE * N_OCC) + r * N_OCC + o

        gcp = pltpu.async_copy(comb_hbm.at[idx_v], g_v, gsem)

        pos_cp.wait()

        def _param_row(r, carry):
            x3 = plsc.load_gather(x_v, [_bcast_i32(r), _bcast_i32(3)])
            for j in range(PARAM_D // L):
                sl = pl.ds(EMB_D + j * L, L)
                pe = x3 * w_v[pl.ds(j * L, L)] + b_v[pl.ds(j * L, L)]
                out_v[r, sl] = pe + out_v[r, sl]
            return carry

        lax.fori_loop(0, C, _param_row, 0)

        gcp.wait()

        def _add_row(r, carry):
            for j in range(EMB_D // L):
                sl = pl.ds(j * L, L)
                out_v[r, sl] = g_v[r, sl] + out_v[r, sl]
            return carry

        lax.fori_loop(0, C, _add_row, 0)

        pltpu.sync_copy(out_v, out_hbm.at[pl.ds(rb, C)])


@jax.jit
def _sc_embed(x, comb_table, pos_table, w_vec, b_vec):
    mesh = plsc.VectorSubcoreMesh(core_axis_name="c", subcore_axis_name="s",
                                  num_cores=NC, num_subcores=NS)
    fn = pl.kernel(
        _sc_body,
        out_type=jax.ShapeDtypeStruct((T, D), jnp.float32),
        mesh=mesh,
        compiler_params=pltpu.CompilerParams(needs_layout_passes=False),
        scratch_types=[
            pltpu.VMEM((C, D), jnp.float32),
            pltpu.VMEM((C, EMB_D), jnp.float32),
            pltpu.VMEM((C, 4), jnp.float32),
            pltpu.VMEM((C,), jnp.int32),
            pltpu.VMEM((PARAM_D,), jnp.float32),
            pltpu.VMEM((PARAM_D,), jnp.float32),
            pltpu.SemaphoreType.DMA,
            pltpu.SemaphoreType.DMA,
        ],
    )
    return fn(x, comb_table, pos_table, w_vec, b_vec)


def kernel(x, gate_table, role_table, occ_table, pos_table, w_param, b_param):
    comb = jnp.concatenate([
        jnp.broadcast_to(gate_table[:, None, None, :],
                         (N_GATE, N_ROLE, N_OCC, GATE_D)),
        jnp.broadcast_to(role_table[None, :, None, :],
                         (N_GATE, N_ROLE, N_OCC, ROLE_D)),
        jnp.broadcast_to(occ_table[None, None, :, :],
                         (N_GATE, N_ROLE, N_OCC, OCC_D)),
    ], axis=-1).reshape(N_GATE * N_ROLE * N_OCC, EMB_D)
    w_vec = w_param.reshape(PARAM_D)
    return _sc_embed(x, comb, pos_table, w_vec, b_param)


# speedup vs baseline: 1.0285x; 1.0285x over previous
import functools

import jax
import jax.numpy as jnp
from jax import lax
from jax.experimental import pallas as pl
from jax.experimental.pallas import tpu as pltpu
from jax.experimental.pallas import tpu_sc as plsc

T = 8192
D = 1024
GATE_D = 512
ROLE_D = 192
OCC_D = 64
PARAM_D = 256
EMB_D = GATE_D + ROLE_D + OCC_D
N_GATE = 20
N_ROLE = 4
N_OCC = 2

NC = 2
NS = 16
NW = NC * NS
L = 16

ROWS_PER_W = T // NW
C = 32
N_CHUNKS = ROWS_PER_W // C


def _bcast_i32(val):
    return jnp.full((L,), val, jnp.int32)


def _sc_body(x_hbm, comb_hbm, pos_hbm, w_hbm, b_hbm,
             out_hbm, out_v, g_v, x_v, idx_v, w_v, b_v, psem, gsem):
    cid = lax.axis_index("c")
    sid = lax.axis_index("s")
    wid = sid * NC + cid
    base = wid * ROWS_PER_W

    pltpu.sync_copy(w_hbm, w_v)
    pltpu.sync_copy(b_hbm, b_v)

    iota = lax.iota(jnp.int32, L)

    def _chunk(k, carry):
        rb = base + k * C
        with jax.named_scope("pos_start"):
            pos_cp = pltpu.async_copy(pos_hbm.at[pl.ds(rb, C)], out_v, psem)
        with jax.named_scope("x_copy"):
            pltpu.sync_copy(x_hbm.at[pl.ds(rb, C)], x_v)

        with jax.named_scope("idx"):
            for j in range(C // L):
                rows = iota + (j * L)
                g = plsc.load_gather(
                    x_v, [rows, _bcast_i32(0)]).astype(jnp.int32)
                r = plsc.load_gather(
                    x_v, [rows, _bcast_i32(1)]).astype(jnp.int32)
                o = plsc.load_gather(
                    x_v, [rows, _bcast_i32(2)]).astype(jnp.int32)
                idx_v[pl.ds(j * L, L)] = g * (N_ROLE * N_OCC) + r * N_OCC + o

        gcp = pltpu.async_copy(comb_hbm.at[idx_v],
                               g_v.at[:, pl.ds(0, EMB_D)], gsem)

        with jax.named_scope("param"):
            @plsc.parallel_loop(0, C, 1, unroll=2)
            def _param_row(r):
                x3 = plsc.load_gather(x_v, [_bcast_i32(r), _bcast_i32(3)])
                for j in range(PARAM_D // L):
                    g_v[r, pl.ds(EMB_D + j * L, L)] = (
                        x3 * w_v[pl.ds(j * L, L)] + b_v[pl.ds(j * L, L)])

        with jax.named_scope("dma_wait"):
            gcp.wait()
            pos_cp.wait()

        with jax.named_scope("add"):
            @plsc.parallel_loop(0, C, 1, unroll=2)
            def _add_row(r):
                for j in range(D // L):
                    sl = pl.ds(j * L, L)
                    plsc.addupdate(out_v.at[r, sl], g_v[r, sl])

        with jax.named_scope("out_copy"):
            pltpu.sync_copy(out_v, out_hbm.at[pl.ds(rb, C)])
        return carry

    lax.fori_loop(0, N_CHUNKS, _chunk, 0)


@jax.jit
def _sc_embed(x, comb_table, pos_table, w_vec, b_vec):
    mesh = plsc.VectorSubcoreMesh(core_axis_name="c", subcore_axis_name="s",
                                  num_cores=NC, num_subcores=NS)
    fn = pl.kernel(
        _sc_body,
        out_type=jax.ShapeDtypeStruct((T, D), jnp.float32),
        mesh=mesh,
        compiler_params=pltpu.CompilerParams(needs_layout_passes=False),
        scratch_types=[
            pltpu.VMEM((C, D), jnp.float32),
            pltpu.VMEM((C, D), jnp.float32),
            pltpu.VMEM((C, 4), jnp.float32),
            pltpu.VMEM((C,), jnp.int32),
            pltpu.VMEM((PARAM_D,), jnp.float32),
            pltpu.VMEM((PARAM_D,), jnp.float32),
            pltpu.SemaphoreType.DMA,
            pltpu.SemaphoreType.DMA,
        ],
    )
    return fn(x, comb_table, pos_table, w_vec, b_vec)


def kernel(x, gate_table, role_table, occ_table, pos_table, w_param, b_param):
    comb = jnp.concatenate([
        jnp.broadcast_to(gate_table[:, None, None, :],
                         (N_GATE, N_ROLE, N_OCC, GATE_D)),
        jnp.broadcast_to(role_table[None, :, None, :],
                         (N_GATE, N_ROLE, N_OCC, ROLE_D)),
        jnp.broadcast_to(occ_table[None, None, :, :],
                         (N_GATE, N_ROLE, N_OCC, OCC_D)),
    ], axis=-1).reshape(N_GATE * N_ROLE * N_OCC, EMB_D)
    w_vec = w_param.reshape(PARAM_D)
    return _sc_embed(x, comb, pos_table, w_vec, b_param)
